# baseline (speedup 1.0000x reference)
import functools

import jax
import jax.numpy as jnp
from jax.experimental import pallas as pl
from jax.experimental.pallas import tpu as pltpu

TEMPERATURE = 1.0
EPS = 1e-20
B, V = 128, 100000

BLK_W = 12544
NBLK = (V + BLK_W - 1) // BLK_W

INT_MAX = 2**31 - 1


def _argmax_kernel(l_ref, u_ref, idx_ref, rmax_ref, ridx_ref):
    j = pl.program_id(0)
    g = -jnp.log(-jnp.log(u_ref[...] + EPS) + EPS)
    s = l_ref[...] + g
    col = jax.lax.broadcasted_iota(jnp.int32, s.shape, 1) + j * BLK_W
    s = jnp.where(col < V, s, -jnp.inf)
    bmax = jnp.max(s, axis=1, keepdims=True)
    bidx = jnp.min(jnp.where(s == bmax, col, INT_MAX), axis=1, keepdims=True)

    @pl.when(j == 0)
    def _init():
        rmax_ref[...] = bmax
        ridx_ref[...] = bidx

    @pl.when(j > 0)
    def _update():
        better = bmax > rmax_ref[...]
        rmax_ref[...] = jnp.where(better, bmax, rmax_ref[...])
        ridx_ref[...] = jnp.where(better, bidx, ridx_ref[...])

    @pl.when(j == NBLK - 1)
    def _emit():
        idx_ref[...] = ridx_ref[...]


def _onehot_kernel(idx_ref, o_ref):
    j = pl.program_id(0)
    shape = o_ref.shape
    col = jax.lax.broadcasted_iota(jnp.int32, shape, 1) + j * shape[1]
    match = col == idx_ref[...]
    anyhot = jnp.any(match, axis=0, keepdims=True)
    row = jax.lax.broadcasted_iota(jnp.int32, shape, 0)
    o_ref[...] = jnp.where((row == 0) & anyhot, 1.0, 0.0).astype(jnp.float32)


@functools.partial(jax.jit, static_argnames=("interpret",))
def kernel(logits, gumbel_u, interpret=False):
    idx = pl.pallas_call(
        _argmax_kernel,
        grid=(NBLK,),
        in_specs=[
            pl.BlockSpec((B, BLK_W), lambda j: (0, j)),
            pl.BlockSpec((B, BLK_W), lambda j: (0, j)),
        ],
        out_specs=pl.BlockSpec((B, 1), lambda j: (0, 0)),
        out_shape=jax.ShapeDtypeStruct((B, 1), jnp.int32),
        scratch_shapes=[
            pltpu.VMEM((B, 1), jnp.float32),
            pltpu.VMEM((B, 1), jnp.int32),
        ],
        interpret=interpret,
    )(logits, gumbel_u)

    out = pl.pallas_call(
        _onehot_kernel,
        grid=(NBLK,),
        in_specs=[pl.BlockSpec((B, 1), lambda j: (0, 0))],
        out_specs=pl.BlockSpec((B, BLK_W), lambda j: (0, j)),
        out_shape=jax.ShapeDtypeStruct((B, V), jnp.float32),
        interpret=interpret,
    )(idx)
    return out
